# baseline (speedup 1.0000x reference)
import jax
import jax.numpy as jnp
from jax.experimental import pallas as pl

_H = 256
_NTYPES = ["cfg", "ast", "test"]
_SIZES = {"cfg": 10000, "ast": 50000, "test": 1000}
_EDGE_META = [("cfg", "cc", "cfg"), ("ast", "aa", "ast"), ("cfg", "ca", "ast"),
              ("ast", "ac", "cfg"), ("test", "tc", "cfg")]


def _softmax_pallas(x):
    n, c = x.shape
    npad = (n + 255) // 256 * 256
    xp = jnp.pad(x, ((0, npad - n), (0, 128 - c)), constant_values=-1e30)

    def body(x_ref, o_ref):
        z = x_ref[...]
        m = jnp.max(z, axis=1, keepdims=True)
        p = jnp.exp(z - m)
        o_ref[...] = p / jnp.sum(p, axis=1, keepdims=True)

    out = pl.pallas_call(
        body,
        grid=(npad // 256,),
        in_specs=[pl.BlockSpec((256, 128), lambda i: (i, 0))],
        out_specs=pl.BlockSpec((256, 128), lambda i: (i, 0)),
        out_shape=jax.ShapeDtypeStruct((npad, 128), jnp.float32),
    )(xp)
    return out[:n, :c]


def _gcn_layer(h, edges, params, li):
    dout = params["l%d_W_cc" % li].shape[1]
    agg = {nt: jnp.zeros((_SIZES[nt], dout), jnp.float32) for nt in _NTYPES}
    deg = {nt: jnp.zeros((_SIZES[nt],), jnp.float32) for nt in _NTYPES}
    for (st, et, dt) in _EDGE_META:
        src, dst = edges[et]
        msg = h[st][src] @ params["l%d_W_%s" % (li, et)]
        agg[dt] = agg[dt] + jax.ops.segment_sum(msg, dst, num_segments=_SIZES[dt])
        deg[dt] = deg[dt] + jax.ops.segment_sum(jnp.ones(dst.shape[0], jnp.float32), dst, num_segments=_SIZES[dt])
    out = {}
    for nt in _NTYPES:
        mean = agg[nt] / jnp.maximum(deg[nt], 1.0)[:, None]
        out[nt] = jax.nn.relu(mean + h[nt] @ params["l%d_self_%s" % (li, nt)] + params["l%d_b_%s" % (li, nt)])
    return out


def kernel(cfg_label, cfg_content, ast_label, ast_arity, ast_content, cc_src, cc_dst, aa_src, aa_dst, ca_src, ca_dst, ac_src, ac_dst, tc_src, tc_dst, params):
    edges = {"cc": (cc_src, cc_dst), "aa": (aa_src, aa_dst), "ca": (ca_src, ca_dst), "ac": (ac_src, ac_dst), "tc": (tc_src, tc_dst)}
    h = {}
    h["cfg"] = jnp.concatenate([params["cfg_label_emb"][cfg_label], cfg_content @ params["cfg_content_W"] + params["cfg_content_b"]], axis=-1)
    h["ast"] = jnp.concatenate([params["ast_label_emb"][ast_label] + params["ast_arity_emb"][ast_arity], ast_content @ params["ast_content_W"] + params["ast_content_b"]], axis=-1)
    h["test"] = jnp.tile(params["test_emb"][None, :], (_SIZES["test"], 1))
    h = _gcn_layer(h, edges, params, 0)
    skip = {nt: h[nt] for nt in _NTYPES}
    h = _gcn_layer(h, edges, params, 1)
    h = {nt: jnp.concatenate([skip[nt], h[nt]], axis=-1) for nt in _NTYPES}
    h = _gcn_layer(h, edges, params, 2)
    skip = {nt: h[nt] for nt in _NTYPES}
    h = _gcn_layer(h, edges, params, 3)
    h = {nt: jnp.concatenate([skip[nt], h[nt]], axis=-1) for nt in _NTYPES}
    h = _gcn_layer(h, edges, params, 4)
    cfg_logits = h["cfg"] @ params["dec_W"] + params["dec_b"]
    ast_logits = h["ast"] @ params["ast_dec_W"] + params["ast_dec_b"]
    return (cfg_logits, _softmax_pallas(cfg_logits), ast_logits, _softmax_pallas(ast_logits))
